# Initial kernel scaffold; baseline (speedup 1.0000x reference)
import jax
import jax.numpy as jnp
from jax.experimental import pallas as pl


def kernel(hidden_states, gate_w, e_score_correction_bias, w_gate_up, w_down, shared_w_gate_up, shared_w_down):
    raise NotImplementedError("write your pallas kernel here")


# speedup vs baseline: 1.3018x; 1.3018x over previous
import functools

import jax
import jax.numpy as jnp
from jax.experimental import pallas as pl
from jax.experimental.pallas import tpu as pltpu

T = 4096
HIDDEN = 1024
E = 8
TOPK = 2
DFF = 512
NGROUP = 4
TOPK_GROUP = 2
ROUTED_SCALING = 2.5
NE_AUG = E + 2

BT = 512


def _routing_combine(h, gate_w, bias_row):
    logits = jnp.dot(h, gate_w, preferred_element_type=jnp.float32)
    s = jax.nn.sigmoid(logits)
    sc = s + bias_row

    gcols = [sc[:, 2 * g:2 * g + 1] + sc[:, 2 * g + 1:2 * g + 2]
             for g in range(NGROUP)]
    keep_group = []
    for g in range(NGROUP):
        rank = jnp.zeros_like(gcols[g], dtype=jnp.int32)
        for g2 in range(NGROUP):
            if g2 == g:
                continue
            if g2 < g:
                beats = gcols[g2] >= gcols[g]
            else:
                beats = gcols[g2] > gcols[g]
            rank = rank + beats.astype(jnp.int32)
        keep_group.append(rank < TOPK_GROUP)

    neg_inf = jnp.float32(-jnp.inf)
    mcols = [jnp.where(keep_group[e // 2], sc[:, e:e + 1], neg_inf)
             for e in range(E)]
    wcols = []
    for e in range(E):
        rank = jnp.zeros((h.shape[0], 1), dtype=jnp.int32)
        for e2 in range(E):
            if e2 == e:
                continue
            if e2 < e:
                beats = mcols[e2] >= mcols[e]
            else:
                beats = mcols[e2] > mcols[e]
            rank = rank + beats.astype(jnp.int32)
        keep = rank < TOPK
        wcols.append(jnp.where(keep, s[:, e:e + 1], 0.0))
    w = jnp.concatenate(wcols, axis=1)
    wsum = jnp.sum(w, axis=1, keepdims=True)
    return w / wsum * ROUTED_SCALING


def _moe_body(hid_ref, gw_ref, bias_ref, wgu_ref, wd_ref, out_ref, comb_ref):
    e = pl.program_id(1)

    @pl.when(e == 0)
    def _():
        combine = _routing_combine(hid_ref[...], gw_ref[...], bias_ref[...])
        comb_ref[:, 0:E] = combine
        comb_ref[:, E:NE_AUG] = jnp.ones((hid_ref.shape[0], 2), jnp.float32)
        out_ref[...] = jnp.zeros_like(out_ref)

    h = hid_ref[...]
    gu = jnp.dot(h, wgu_ref[0], preferred_element_type=jnp.float32)
    g = gu[:, :DFF]
    u = gu[:, DFF:]
    act = g * jax.nn.sigmoid(g) * u
    o = jnp.dot(act, wd_ref[0], preferred_element_type=jnp.float32)

    lane = jax.lax.broadcasted_iota(jnp.int32, (h.shape[0], NE_AUG), 1)
    col = jnp.sum(jnp.where(lane == e, comb_ref[...], 0.0),
                  axis=1, keepdims=True)
    out_ref[...] += col * o


@functools.partial(jax.jit, static_argnames=("interpret",))
def _fused_moe(hidden_states, gate_w, bias_row, wgu_aug, wd_aug,
               interpret=False):
    grid = (T // BT, NE_AUG)
    return pl.pallas_call(
        _moe_body,
        grid=grid,
        in_specs=[
            pl.BlockSpec((BT, HIDDEN), lambda i, e: (i, 0)),
            pl.BlockSpec((HIDDEN, E), lambda i, e: (0, 0)),
            pl.BlockSpec((1, E), lambda i, e: (0, 0)),
            pl.BlockSpec((1, HIDDEN, 2 * DFF), lambda i, e: (e, 0, 0)),
            pl.BlockSpec((1, DFF, HIDDEN), lambda i, e: (e, 0, 0)),
        ],
        out_specs=pl.BlockSpec((BT, HIDDEN), lambda i, e: (i, 0)),
        out_shape=jax.ShapeDtypeStruct((T, HIDDEN), jnp.float32),
        scratch_shapes=[pltpu.VMEM((BT, NE_AUG), jnp.float32)],
        interpret=interpret,
    )(hidden_states, gate_w, bias_row, wgu_aug, wd_aug)


def kernel(hidden_states, gate_w, e_score_correction_bias, w_gate_up, w_down,
           shared_w_gate_up, shared_w_down, interpret=False):
    sg = shared_w_gate_up[:, :2 * DFF]
    su = shared_w_gate_up[:, 2 * DFF:]
    pseudo_a = jnp.concatenate([sg[:, :DFF], su[:, :DFF]], axis=1)
    pseudo_b = jnp.concatenate([sg[:, DFF:], su[:, DFF:]], axis=1)
    wgu_aug = jnp.concatenate([w_gate_up, pseudo_a[None], pseudo_b[None]],
                              axis=0)
    wd_aug = jnp.concatenate(
        [w_down, shared_w_down[None, :DFF], shared_w_down[None, DFF:]], axis=0)
    bias_row = e_score_correction_bias.reshape(1, E)
    return _fused_moe(hidden_states, gate_w, bias_row, wgu_aug, wd_aug,
                      interpret=interpret)


# speedup vs baseline: 1.5492x; 1.1900x over previous
import functools

import jax
import jax.numpy as jnp
from jax.experimental import pallas as pl
from jax.experimental.pallas import tpu as pltpu

T = 4096
HIDDEN = 1024
E = 8
TOPK = 2
DFF = 512
NGROUP = 4
TOPK_GROUP = 2
ROUTED_SCALING = 2.5
NE_AUG = E + 2

BT = 512


def _routing_combine(h, gate_w, bias_row):
    logits = jnp.dot(h, gate_w, preferred_element_type=jnp.float32)
    s = jax.nn.sigmoid(logits)
    sc = s + bias_row

    gcols = [sc[:, 2 * g:2 * g + 1] + sc[:, 2 * g + 1:2 * g + 2]
             for g in range(NGROUP)]
    keep_group = []
    for g in range(NGROUP):
        rank = jnp.zeros_like(gcols[g], dtype=jnp.int32)
        for g2 in range(NGROUP):
            if g2 == g:
                continue
            if g2 < g:
                beats = gcols[g2] >= gcols[g]
            else:
                beats = gcols[g2] > gcols[g]
            rank = rank + beats.astype(jnp.int32)
        keep_group.append(rank < TOPK_GROUP)

    neg_inf = jnp.float32(-jnp.inf)
    mcols = [jnp.where(keep_group[e // 2], sc[:, e:e + 1], neg_inf)
             for e in range(E)]
    wcols = []
    for e in range(E):
        rank = jnp.zeros((h.shape[0], 1), dtype=jnp.int32)
        for e2 in range(E):
            if e2 == e:
                continue
            if e2 < e:
                beats = mcols[e2] >= mcols[e]
            else:
                beats = mcols[e2] > mcols[e]
            rank = rank + beats.astype(jnp.int32)
        keep = rank < TOPK
        wcols.append(jnp.where(keep, s[:, e:e + 1], 0.0))
    w = jnp.concatenate(wcols, axis=1)
    wsum = jnp.sum(w, axis=1, keepdims=True)
    return w / wsum * ROUTED_SCALING


def _moe_body(hid_ref, gw_ref, bias_ref, wgu_ref, wd_ref, out_ref, comb_ref):
    e = pl.program_id(1)

    @pl.when(e == 0)
    def _():
        combine = _routing_combine(hid_ref[...], gw_ref[...], bias_ref[...])
        comb_ref[:, 0:E] = combine
        comb_ref[:, E:NE_AUG] = jnp.ones((hid_ref.shape[0], 2), jnp.float32)
        out_ref[...] = jnp.zeros_like(out_ref)

    h = hid_ref[...].astype(jnp.bfloat16)
    gu = jnp.dot(h, wgu_ref[0], preferred_element_type=jnp.float32)
    g = gu[:, :DFF]
    u = gu[:, DFF:]
    act = (g * jax.nn.sigmoid(g) * u).astype(jnp.bfloat16)
    o = jnp.dot(act, wd_ref[0], preferred_element_type=jnp.float32)

    lane = jax.lax.broadcasted_iota(jnp.int32, (h.shape[0], NE_AUG), 1)
    col = jnp.sum(jnp.where(lane == e, comb_ref[...], 0.0),
                  axis=1, keepdims=True)
    out_ref[...] += col * o


@functools.partial(jax.jit, static_argnames=("interpret",))
def _fused_moe(hidden_states, gate_w, bias_row, wgu_aug, wd_aug,
               interpret=False):
    grid = (T // BT, NE_AUG)
    return pl.pallas_call(
        _moe_body,
        grid=grid,
        in_specs=[
            pl.BlockSpec((BT, HIDDEN), lambda i, e: (i, 0)),
            pl.BlockSpec((HIDDEN, E), lambda i, e: (0, 0)),
            pl.BlockSpec((1, E), lambda i, e: (0, 0)),
            pl.BlockSpec((1, HIDDEN, 2 * DFF), lambda i, e: (e, 0, 0)),
            pl.BlockSpec((1, DFF, HIDDEN), lambda i, e: (e, 0, 0)),
        ],
        out_specs=pl.BlockSpec((BT, HIDDEN), lambda i, e: (i, 0)),
        out_shape=jax.ShapeDtypeStruct((T, HIDDEN), jnp.float32),
        scratch_shapes=[pltpu.VMEM((BT, NE_AUG), jnp.float32)],
        interpret=interpret,
    )(hidden_states, gate_w, bias_row, wgu_aug, wd_aug)


def kernel(hidden_states, gate_w, e_score_correction_bias, w_gate_up, w_down,
           shared_w_gate_up, shared_w_down, interpret=False):
    sg = shared_w_gate_up[:, :2 * DFF]
    su = shared_w_gate_up[:, 2 * DFF:]
    pseudo_a = jnp.concatenate([sg[:, :DFF], su[:, :DFF]], axis=1)
    pseudo_b = jnp.concatenate([sg[:, DFF:], su[:, DFF:]], axis=1)
    wgu_aug = jnp.concatenate([w_gate_up, pseudo_a[None], pseudo_b[None]],
                              axis=0).astype(jnp.bfloat16)
    wd_aug = jnp.concatenate(
        [w_down, shared_w_down[None, :DFF], shared_w_down[None, DFF:]],
        axis=0).astype(jnp.bfloat16)
    bias_row = e_score_correction_bias.reshape(1, E)
    return _fused_moe(hidden_states, gate_w, bias_row, wgu_aug, wd_aug,
                      interpret=interpret)
